# baseline (device time: 97103 ns/iter reference)
import jax
import jax.numpy as jnp
from jax import lax
from jax.experimental import pallas as pl
from jax.experimental.pallas import tpu as pltpu

N_DEV = 4
E4M3 = jnp.float8_e4m3fn
E5M2 = jnp.float8_e5m2


def kernel(x, w_mat, scale_x, scale_w):
    m_total, k_per = x.shape
    _, n = w_mat.shape
    m_per = m_total // N_DEV

    scale = (scale_x.astype(jnp.float32) * scale_w.astype(jnp.float32)).reshape(1, 1)

    def body(scale_ref, x_ref, w_ref, out_ref,
             w8, x8, wrecv, xrecv,
             send_sems, wrecv_sems, xrecv_sems, dummy_sem):
        my = lax.axis_index("i")
        left = lax.rem(my + N_DEV - 1, N_DEV)
        right = lax.rem(my + 1, N_DEV)
        diag = lax.rem(my + 2, N_DEV)

        barrier = pltpu.get_barrier_semaphore()
        for nbr in (left, right, diag):
            pl.semaphore_signal(
                barrier, inc=1,
                device_id=(nbr,), device_id_type=pl.DeviceIdType.MESH,
            )
        pl.semaphore_wait(barrier, 3)

        sc = scale_ref[0, 0]
        sends = []

        def start(src, dst, recv_sem, sem_i, tgt):
            d = pltpu.make_async_remote_copy(
                src_ref=src, dst_ref=dst,
                send_sem=send_sems.at[sem_i], recv_sem=recv_sem,
                device_id=(tgt,), device_id_type=pl.DeviceIdType.MESH,
            )
            d.start()
            sends.append(d)

        def wait_recv(buf_slot, recv_sem, src_dev):
            d = pltpu.make_async_remote_copy(
                src_ref=buf_slot, dst_ref=buf_slot,
                send_sem=dummy_sem.at[0], recv_sem=recv_sem,
                device_id=(src_dev,), device_id_type=pl.DeviceIdType.MESH,
            )
            d.wait_recv()

        w8[...] = w_ref[...].astype(E5M2)
        start(w8, wrecv.at[0], wrecv_sems.at[0], 0, right)
        start(w8, wrecv.at[1], wrecv_sems.at[1], 1, left)
        start(w8, wrecv.at[2], wrecv_sems.at[2], 2, diag)

        x8[0] = x_ref[pl.ds(right * m_per, m_per), :].astype(E4M3)
        start(x8.at[0], xrecv.at[0], xrecv_sems.at[0], 3, right)
        x8[1] = x_ref[pl.ds(left * m_per, m_per), :].astype(E4M3)
        start(x8.at[1], xrecv.at[1], xrecv_sems.at[1], 4, left)
        x8[2] = x_ref[pl.ds(diag * m_per, m_per), :].astype(E4M3)
        start(x8.at[2], xrecv.at[2], xrecv_sems.at[2], 5, diag)

        w_bf = w_ref[...].astype(jnp.bfloat16)
        xa = x_ref[pl.ds(my * m_per, m_per), :].astype(jnp.bfloat16)
        out_ref[...] = jnp.dot(xa, w_bf, preferred_element_type=jnp.float32)

        for slot, src_dev in ((0, left), (1, right), (2, diag)):
            wait_recv(wrecv.at[slot], wrecv_sems.at[slot], src_dev)
            wait_recv(xrecv.at[slot], xrecv_sems.at[slot], src_dev)
            d = jnp.dot(
                xrecv[slot].astype(jnp.bfloat16),
                wrecv[slot].astype(jnp.bfloat16),
                preferred_element_type=jnp.float32,
            )
            out_ref[...] = out_ref[...] + d

        y = out_ref[...] * sc
        out_ref[...] = y * jax.nn.sigmoid(y)

        for d in sends:
            d.wait_send()

    return pl.pallas_call(
        body,
        out_shape=jax.ShapeDtypeStruct((m_per, n), jnp.float32),
        in_specs=[
            pl.BlockSpec(memory_space=pltpu.SMEM),
            pl.BlockSpec(memory_space=pltpu.VMEM),
            pl.BlockSpec(memory_space=pltpu.VMEM),
        ],
        out_specs=pl.BlockSpec(memory_space=pltpu.VMEM),
        scratch_shapes=[
            pltpu.VMEM((k_per, n), E5M2),
            pltpu.VMEM((3, m_per, k_per), E4M3),
            pltpu.VMEM((3, k_per, n), E5M2),
            pltpu.VMEM((3, m_per, k_per), E4M3),
            pltpu.SemaphoreType.DMA((6,)),
            pltpu.SemaphoreType.DMA((3,)),
            pltpu.SemaphoreType.DMA((3,)),
            pltpu.SemaphoreType.DMA((1,)),
        ],
        compiler_params=pltpu.CompilerParams(
            collective_id=0,
            vmem_limit_bytes=100 * 1024 * 1024,
        ),
    )(scale, x, w_mat)
